# baseline (speedup 1.0000x reference)
import functools

import jax
import jax.numpy as jnp
from jax import lax
from jax.experimental import pallas as pl
from jax.experimental.pallas import tpu as pltpu
from jax.experimental.pallas import tpu_sc as plsc

N = 10000
E = 320000
D = 128

NC = 2
NS = 16
NW = NC * NS

GCHUNK = 80
SCHUNK = 80
NP = 10240
ROWS_PER_SUB = NP // NS

_f32 = jnp.float32


GSTEPS = (2 * E) // NW // GCHUNK
SSTEPS = E // NW // SCHUNK


def _gather_body(table, idx3, out_hbm, idx_v, rows0, rows1,
                 sg0, sg1, sw0, sw1):
    c = lax.axis_index("c")
    s = lax.axis_index("s")
    wid = c * NS + s
    base = wid * GSTEPS * GCHUNK
    pltpu.sync_copy(idx3.at[wid], idx_v)

    def pair(j, carry):
        k0 = 2 * j
        k1 = k0 + 1
        g0 = pltpu.async_copy(table.at[idx_v.at[k0]], rows0, sg0)
        g1 = pltpu.async_copy(table.at[idx_v.at[k1]], rows1, sg1)
        g0.wait()
        w0 = pltpu.async_copy(
            rows0, out_hbm.at[pl.ds(base + k0 * GCHUNK, GCHUNK)], sw0)
        g1.wait()
        w1 = pltpu.async_copy(
            rows1, out_hbm.at[pl.ds(base + k1 * GCHUNK, GCHUNK)], sw1)
        w0.wait()
        w1.wait()
        return carry

    lax.fori_loop(0, GSTEPS // 2, pair, 0)


def _sc_gather(node_feat, idx3):
    return pl.kernel(
        _gather_body,
        out_type=jax.ShapeDtypeStruct((2 * E, D), _f32),
        mesh=plsc.VectorSubcoreMesh(core_axis_name="c", subcore_axis_name="s"),
        scratch_types=[
            pltpu.VMEM((GSTEPS, GCHUNK), jnp.int32),
            pltpu.VMEM((GCHUNK, D), _f32),
            pltpu.VMEM((GCHUNK, D), _f32),
            pltpu.SemaphoreType.DMA,
            pltpu.SemaphoreType.DMA,
            pltpu.SemaphoreType.DMA,
            pltpu.SemaphoreType.DMA,
        ],
    )(node_feat, idx3)


def _scatter_body(mess, dst3, zinit, out_hbm, idx_v, rows0, rows1, acc,
                  sl0, sl1, ss0, ss1):
    c = lax.axis_index("c")
    s = lax.axis_index("s")

    @pl.when(s == 0)
    def _init():
        pltpu.sync_copy(zinit, acc)

    plsc.subcore_barrier()

    wid = c * NS + s
    base = wid * SSTEPS * SCHUNK
    pltpu.sync_copy(dst3.at[wid], idx_v)

    def pair(j, carry):
        k0 = 2 * j
        k1 = k0 + 1
        l0 = pltpu.async_copy(
            mess.at[pl.ds(base + k0 * SCHUNK, SCHUNK)], rows0, sl0)
        l1 = pltpu.async_copy(
            mess.at[pl.ds(base + k1 * SCHUNK, SCHUNK)], rows1, sl1)
        l0.wait()
        s0 = pltpu.async_copy(rows0, acc.at[idx_v.at[k0]], ss0, add=True)
        l1.wait()
        s1 = pltpu.async_copy(rows1, acc.at[idx_v.at[k1]], ss1, add=True)
        s0.wait()
        s1.wait()
        return carry

    lax.fori_loop(0, SSTEPS // 2, pair, 0)
    kt = SSTEPS - 1
    pltpu.sync_copy(mess.at[pl.ds(base + kt * SCHUNK, SCHUNK)], rows0)
    pltpu.sync_copy(rows0, acc.at[idx_v.at[kt]], add=True)

    plsc.subcore_barrier()
    rbase = s * ROWS_PER_SUB
    pltpu.sync_copy(acc.at[pl.ds(rbase, ROWS_PER_SUB)],
                    out_hbm.at[c, pl.ds(rbase, ROWS_PER_SUB)])


def _sc_scatter(mess, dst3, zinit):
    return pl.kernel(
        _scatter_body,
        out_type=jax.ShapeDtypeStruct((NC, NP, D), _f32),
        mesh=plsc.VectorSubcoreMesh(core_axis_name="c", subcore_axis_name="s"),
        scratch_types=[
            pltpu.VMEM((SSTEPS, SCHUNK), jnp.int32),
            pltpu.VMEM((SCHUNK, D), _f32),
            pltpu.VMEM((SCHUNK, D), _f32),
            pltpu.VMEM_SHARED((NP, D), _f32),
            pltpu.SemaphoreType.DMA,
            pltpu.SemaphoreType.DMA,
            pltpu.SemaphoreType.DMA,
            pltpu.SemaphoreType.DMA,
        ],
    )(mess, dst3, zinit)


def _edge_body(vi, vj, ef, rbf, wsrc, wdst, wee, wen, w2, bias, rbfw,
               new_e, mess):
    f32 = jnp.float32
    bf16 = jnp.bfloat16
    efb = ef[:]
    pvi = jnp.dot(vi[:].astype(bf16), wsrc[:], preferred_element_type=f32)
    pvj = jnp.dot(vj[:].astype(bf16), wdst[:], preferred_element_type=f32)
    basep = pvi + pvj
    pe = jnp.dot(efb.astype(bf16), wee[:], preferred_element_type=f32)
    r = jnp.dot(rbf[:], rbfw[:], preferred_element_type=f32)

    e_h1 = jax.nn.silu(basep[:, 0:D] + pe[:, 0:D] + bias[0])
    e_g1 = jax.nn.silu(basep[:, D:2 * D] + pe[:, D:2 * D] + bias[2])
    e_h2 = jax.nn.silu(
        jnp.dot(e_h1.astype(bf16), w2[0], preferred_element_type=f32)
        + bias[1])
    e_g = jax.nn.sigmoid(
        jnp.dot(e_g1.astype(bf16), w2[1], preferred_element_type=f32)
        + bias[3])
    ne = efb + e_h2 * e_g * r[:, 0:D]
    new_e[:] = ne

    pne = jnp.dot(ne.astype(bf16), wen[:], preferred_element_type=f32)
    n_h1 = jax.nn.silu(basep[:, 2 * D:3 * D] + pne[:, 0:D] + bias[4])
    n_g1 = jax.nn.silu(basep[:, 3 * D:4 * D] + pne[:, D:2 * D] + bias[6])
    n_h2 = jax.nn.silu(
        jnp.dot(n_h1.astype(bf16), w2[2], preferred_element_type=f32)
        + bias[5])
    n_g = jax.nn.sigmoid(
        jnp.dot(n_g1.astype(bf16), w2[3], preferred_element_type=f32)
        + bias[7])
    mess[:] = n_h2 * n_g * r[:, D:2 * D]


def _tc_edge(vi, vj, ef, rbfp, wsrc, wdst, wee, wen, w2, bias, rbfw, blk):
    grid = (E // blk,)
    row = lambda i: (i, 0)
    whole2 = lambda i: (0, 0)
    whole3 = lambda i: (0, 0, 0)
    return pl.pallas_call(
        _edge_body,
        grid=grid,
        in_specs=[
            pl.BlockSpec((blk, D), row),
            pl.BlockSpec((blk, D), row),
            pl.BlockSpec((blk, D), row),
            pl.BlockSpec((blk, 16), row),
            pl.BlockSpec((D, 4 * D), whole2),
            pl.BlockSpec((D, 4 * D), whole2),
            pl.BlockSpec((D, 2 * D), whole2),
            pl.BlockSpec((D, 2 * D), whole2),
            pl.BlockSpec((4, D, D), whole3),
            pl.BlockSpec((8, D), whole2),
            pl.BlockSpec((16, 2 * D), whole2),
        ],
        out_specs=[
            pl.BlockSpec((blk, D), row),
            pl.BlockSpec((blk, D), row),
        ],
        out_shape=[
            jax.ShapeDtypeStruct((E, D), _f32),
            jax.ShapeDtypeStruct((E, D), _f32),
        ],
        compiler_params=pltpu.CompilerParams(
            dimension_semantics=("arbitrary",)),
    )(vi, vj, ef, rbfp, wsrc, wdst, wee, wen, w2, bias, rbfw)


def _combine_body(nf, p, out):
    out[:] = nf[:] + p[0] + p[1]


def _tc_combine(node_feat, partials):
    blk = 1000
    return pl.pallas_call(
        _combine_body,
        grid=(N // blk,),
        in_specs=[
            pl.BlockSpec((blk, D), lambda i: (i, 0)),
            pl.BlockSpec((NC, blk, D), lambda i: (0, i, 0)),
        ],
        out_specs=pl.BlockSpec((blk, D), lambda i: (i, 0)),
        out_shape=jax.ShapeDtypeStruct((N, D), _f32),
    )(node_feat, partials)


def kernel(node_feat, edge_feat, rbf, state_feat, edge_index,
           ew1, eb1, ew2, eb2, egw1, egb1, egw2, egb2, edge_rbf_w,
           nw1, nb1, nw2, nb2, ngw1, ngb1, ngw2, ngb2, node_rbf_w):
    idx_all = edge_index.reshape(2 * E).astype(jnp.int32)
    src_dst_rows = _sc_gather(node_feat,
                              idx_all.reshape(NW, GSTEPS, GCHUNK))
    vi = src_dst_rows[:E]
    vj = src_dst_rows[E:]

    bf16 = jnp.bfloat16
    wsrc = jnp.concatenate(
        [ew1[:D], egw1[:D], nw1[:D], ngw1[:D]], axis=1).astype(bf16)
    wdst = jnp.concatenate(
        [ew1[D:2 * D], egw1[D:2 * D], nw1[D:2 * D], ngw1[D:2 * D]],
        axis=1).astype(bf16)
    wee = jnp.concatenate([ew1[2 * D:], egw1[2 * D:]], axis=1).astype(bf16)
    wen = jnp.concatenate([nw1[2 * D:], ngw1[2 * D:]], axis=1).astype(bf16)
    w2 = jnp.stack([ew2, egw2, nw2, ngw2]).astype(bf16)
    bias = jnp.stack([eb1, eb2, egb1, egb2, nb1, nb2, ngb1, ngb2])
    rbfw = jnp.concatenate(
        [jnp.pad(edge_rbf_w, ((0, 16 - rbf.shape[1]), (0, 0))),
         jnp.pad(node_rbf_w, ((0, 16 - rbf.shape[1]), (0, 0)))], axis=1)
    rbfp = jnp.pad(rbf, ((0, 0), (0, 16 - rbf.shape[1])))

    new_e, mess = _tc_edge(vi, vj, edge_feat, rbfp,
                           wsrc, wdst, wee, wen, w2, bias, rbfw, blk=2000)

    dst = idx_all[E:].reshape(NW, SSTEPS, SCHUNK)
    zinit = jnp.zeros((NP, D), _f32)
    partials = _sc_scatter(mess, dst, zinit)
    new_v = _tc_combine(node_feat, partials)
    return new_e, new_v, state_feat


# speedup vs baseline: 1.1925x; 1.1925x over previous
import jax
import jax.numpy as jnp
from jax import lax
from jax.experimental import pallas as pl
from jax.experimental.pallas import tpu as pltpu
from jax.experimental.pallas import tpu_sc as plsc

N = 10000
E = 320000
D = 128

NC = 2
NS = 16
NW = NC * NS

CH = 5
ECH = E // CH

GCHUNK = 80
SCHUNK = 80
GSTEPS = 2 * ECH // NW // GCHUNK
SA_CH = 4
SA_STEPS = SA_CH * ECH // NW // SCHUNK
SB_STEPS = ECH // NW // SCHUNK

NP = 10240
ROWS_PER_SUB = NP // NS

BLK = 2000
NBLK = ECH // BLK

_f32 = jnp.float32


def _gather_body(table, idx3, out_hbm, idx_v, rows0, rows1,
                 sg0, sg1, sw0, sw1):
    c = lax.axis_index("c")
    s = lax.axis_index("s")
    wid = c * NS + s
    base = wid * GSTEPS * GCHUNK
    pltpu.sync_copy(idx3.at[wid], idx_v)

    def pair(j, carry):
        k0 = 2 * j
        k1 = k0 + 1
        g0 = pltpu.async_copy(table.at[idx_v.at[k0]], rows0, sg0)
        g1 = pltpu.async_copy(table.at[idx_v.at[k1]], rows1, sg1)
        g0.wait()
        w0 = pltpu.async_copy(
            rows0, out_hbm.at[pl.ds(base + k0 * GCHUNK, GCHUNK)], sw0)
        g1.wait()
        w1 = pltpu.async_copy(
            rows1, out_hbm.at[pl.ds(base + k1 * GCHUNK, GCHUNK)], sw1)
        w0.wait()
        w1.wait()
        return carry

    lax.fori_loop(0, GSTEPS // 2, pair, 0)


def _sc_gather(node_feat, idx3):
    return pl.kernel(
        _gather_body,
        out_type=jax.ShapeDtypeStruct((2 * ECH, D), _f32),
        mesh=plsc.VectorSubcoreMesh(core_axis_name="c", subcore_axis_name="s"),
        scratch_types=[
            pltpu.VMEM((GSTEPS, GCHUNK), jnp.int32),
            pltpu.VMEM((GCHUNK, D), _f32),
            pltpu.VMEM((GCHUNK, D), _f32),
            pltpu.SemaphoreType.DMA,
            pltpu.SemaphoreType.DMA,
            pltpu.SemaphoreType.DMA,
            pltpu.SemaphoreType.DMA,
        ],
    )(node_feat, idx3)


def _make_scatter_body(steps, nmess, mess_rows_per_worker):
    pairs = steps // 2
    has_tail = steps % 2 == 1

    def body(*refs):
        mess_refs = refs[:nmess]
        (dst3, init, out_hbm, idx_v, rows0, rows1, acc,
         sl0, sl1, ss0, ss1) = refs[nmess:]
        c = lax.axis_index("c")
        s = lax.axis_index("s")

        @pl.when(s == 0)
        def _init():
            pltpu.sync_copy(init.at[c], acc)

        plsc.subcore_barrier()

        wid = c * NS + s
        pltpu.sync_copy(dst3.at[wid], idx_v)

        def accum(mref, lbase):
            def pair(j, carry):
                k0 = 2 * j
                k1 = k0 + 1
                l0 = pltpu.async_copy(
                    mref.at[pl.ds(lbase + k0 * SCHUNK, SCHUNK)], rows0, sl0)
                l1 = pltpu.async_copy(
                    mref.at[pl.ds(lbase + k1 * SCHUNK, SCHUNK)], rows1, sl1)
                l0.wait()
                s0 = pltpu.async_copy(rows0, acc.at[idx_v.at[k0]], ss0,
                                      add=True)
                l1.wait()
                s1 = pltpu.async_copy(rows1, acc.at[idx_v.at[k1]], ss1,
                                      add=True)
                s0.wait()
                s1.wait()
                return carry

            lax.fori_loop(0, pairs, pair, 0)
            if has_tail:
                kt = steps - 1
                pltpu.sync_copy(
                    mref.at[pl.ds(lbase + kt * SCHUNK, SCHUNK)], rows0)
                pltpu.sync_copy(rows0, acc.at[idx_v.at[kt]], add=True)

        if nmess == 1:
            accum(mess_refs[0], wid * mess_rows_per_worker)
        else:
            wpm = NW // nmess
            sub = wid // wpm
            lbase = (wid % wpm) * mess_rows_per_worker
            for mi in range(nmess):
                @pl.when(sub == mi)
                def _go(mi=mi):
                    accum(mess_refs[mi], lbase)

        plsc.subcore_barrier()
        rbase = s * ROWS_PER_SUB
        pltpu.sync_copy(acc.at[pl.ds(rbase, ROWS_PER_SUB)],
                        out_hbm.at[c, pl.ds(rbase, ROWS_PER_SUB)])

    return body


def _sc_scatter(mess_list, dst3, init, steps):
    nmess = len(mess_list)
    rows_per_worker = nmess * ECH // NW
    return pl.kernel(
        _make_scatter_body(steps, nmess, rows_per_worker),
        out_type=jax.ShapeDtypeStruct((NC, NP, D), _f32),
        mesh=plsc.VectorSubcoreMesh(core_axis_name="c", subcore_axis_name="s"),
        scratch_types=[
            pltpu.VMEM((steps, SCHUNK), jnp.int32),
            pltpu.VMEM((SCHUNK, D), _f32),
            pltpu.VMEM((SCHUNK, D), _f32),
            pltpu.VMEM_SHARED((NP, D), _f32),
            pltpu.SemaphoreType.DMA,
            pltpu.SemaphoreType.DMA,
            pltpu.SemaphoreType.DMA,
            pltpu.SemaphoreType.DMA,
        ],
    )(*mess_list, dst3, init)


def _edge_body(vi, vj, ef, rbf, wsrc, wdst, wee, wen, w2, bias, rbfw,
               new_e, mess):
    f32 = jnp.float32
    efb = ef[:]
    pvi = jnp.dot(vi[:], wsrc[:], preferred_element_type=f32)
    pvj = jnp.dot(vj[:], wdst[:], preferred_element_type=f32)
    basep = pvi + pvj
    pe = jnp.dot(efb, wee[:], preferred_element_type=f32)
    r = jnp.dot(rbf[:], rbfw[:], preferred_element_type=f32)

    e_h1 = jax.nn.silu(basep[:, 0:D] + pe[:, 0:D] + bias[0])
    e_g1 = jax.nn.silu(basep[:, D:2 * D] + pe[:, D:2 * D] + bias[2])
    e_h2 = jax.nn.silu(jnp.dot(e_h1, w2[0], preferred_element_type=f32)
                       + bias[1])
    e_g = jax.nn.sigmoid(jnp.dot(e_g1, w2[1], preferred_element_type=f32)
                         + bias[3])
    ne = efb + e_h2 * e_g * r[:, 0:D]
    new_e[:] = ne

    pne = jnp.dot(ne, wen[:], preferred_element_type=f32)
    n_h1 = jax.nn.silu(basep[:, 2 * D:3 * D] + pne[:, 0:D] + bias[4])
    n_g1 = jax.nn.silu(basep[:, 3 * D:4 * D] + pne[:, D:2 * D] + bias[6])
    n_h2 = jax.nn.silu(jnp.dot(n_h1, w2[2], preferred_element_type=f32)
                       + bias[5])
    n_g = jax.nn.sigmoid(jnp.dot(n_g1, w2[3], preferred_element_type=f32)
                         + bias[7])
    mess[:] = n_h2 * n_g * r[:, D:2 * D]


def _edge_body_alias(ne_in, vi, vj, ef, rbf, wsrc, wdst, wee, wen, w2,
                     bias, rbfw, new_e, mess):
    del ne_in
    _edge_body(vi, vj, ef, rbf, wsrc, wdst, wee, wen, w2, bias, rbfw,
               new_e, mess)


def _weight_specs():
    whole2 = lambda i: (0, 0)
    whole3 = lambda i: (0, 0, 0)
    return [
        pl.BlockSpec((D, 4 * D), whole2),
        pl.BlockSpec((D, 4 * D), whole2),
        pl.BlockSpec((D, 2 * D), whole2),
        pl.BlockSpec((D, 2 * D), whole2),
        pl.BlockSpec((4, D, D), whole3),
        pl.BlockSpec((8, D), whole2),
        pl.BlockSpec((16, 2 * D), whole2),
    ]


def _tc_edge_chunk(ne_buf, vi, vj, ef, rbfp, weights, ch):
    row = lambda i: (i, 0)
    rowc = lambda i, ch=ch: (ch * NBLK + i, 0)
    first = ne_buf is None
    body = _edge_body if first else _edge_body_alias
    in_specs = [
        pl.BlockSpec((BLK, D), row),
        pl.BlockSpec((BLK, D), row),
        pl.BlockSpec((BLK, D), rowc),
        pl.BlockSpec((BLK, 16), rowc),
    ] + _weight_specs()
    args = [vi, vj, ef, rbfp] + list(weights)
    aliases = {}
    if not first:
        in_specs = [pl.BlockSpec(memory_space=pltpu.MemorySpace.HBM)] \
            + in_specs
        args = [ne_buf] + args
        aliases = {0: 0}
    return pl.pallas_call(
        body,
        grid=(NBLK,),
        in_specs=in_specs,
        out_specs=[
            pl.BlockSpec((BLK, D), rowc),
            pl.BlockSpec((BLK, D), row),
        ],
        out_shape=[
            jax.ShapeDtypeStruct((E, D), _f32),
            jax.ShapeDtypeStruct((ECH, D), _f32),
        ],
        input_output_aliases=aliases,
        compiler_params=pltpu.CompilerParams(
            dimension_semantics=("arbitrary",)),
    )(*args)


def _combine_body(nf, p, out):
    out[:] = nf[:] + p[0] + p[1]


def _tc_combine(node_feat, partials):
    blk = 1000
    return pl.pallas_call(
        _combine_body,
        grid=(N // blk,),
        in_specs=[
            pl.BlockSpec((blk, D), lambda i: (i, 0)),
            pl.BlockSpec((NC, blk, D), lambda i: (0, i, 0)),
        ],
        out_specs=pl.BlockSpec((blk, D), lambda i: (i, 0)),
        out_shape=jax.ShapeDtypeStruct((N, D), _f32),
    )(node_feat, partials)


def kernel(node_feat, edge_feat, rbf, state_feat, edge_index,
           ew1, eb1, ew2, eb2, egw1, egb1, egw2, egb2, edge_rbf_w,
           nw1, nb1, nw2, nb2, ngw1, ngb1, ngw2, ngb2, node_rbf_w):
    src = edge_index[0].astype(jnp.int32)
    dst = edge_index[1].astype(jnp.int32)
    gidx = jnp.concatenate(
        [src.reshape(CH, ECH), dst.reshape(CH, ECH)], axis=1)
    gidx = gidx.reshape(CH, NW, GSTEPS, GCHUNK)

    wsrc = jnp.concatenate(
        [ew1[:D], egw1[:D], nw1[:D], ngw1[:D]], axis=1)
    wdst = jnp.concatenate(
        [ew1[D:2 * D], egw1[D:2 * D], nw1[D:2 * D], ngw1[D:2 * D]], axis=1)
    wee = jnp.concatenate([ew1[2 * D:], egw1[2 * D:]], axis=1)
    wen = jnp.concatenate([nw1[2 * D:], ngw1[2 * D:]], axis=1)
    w2 = jnp.stack([ew2, egw2, nw2, ngw2])
    bias = jnp.stack([eb1, eb2, egb1, egb2, nb1, nb2, ngb1, ngb2])
    rbfw = jnp.concatenate(
        [jnp.pad(edge_rbf_w, ((0, 16 - rbf.shape[1]), (0, 0))),
         jnp.pad(node_rbf_w, ((0, 16 - rbf.shape[1]), (0, 0)))], axis=1)
    rbfp = jnp.pad(rbf, ((0, 0), (0, 16 - rbf.shape[1])))
    weights = (wsrc, wdst, wee, wen, w2, bias, rbfw)

    ne_buf = None
    mess_chunks = []
    for ch in range(CH):
        vivj = _sc_gather(node_feat, gidx[ch])
        ne_buf, m = _tc_edge_chunk(ne_buf, vivj[:ECH], vivj[ECH:],
                                   edge_feat, rbfp, weights, ch)
        mess_chunks.append(m)

    dsta = dst[:SA_CH * ECH].reshape(NW, SA_STEPS, SCHUNK)
    dstb = dst[SA_CH * ECH:].reshape(NW, SB_STEPS, SCHUNK)
    zinit = jnp.zeros((NC, NP, D), _f32)
    pa = _sc_scatter(mess_chunks[:SA_CH], dsta, zinit, SA_STEPS)
    pb = _sc_scatter(mess_chunks[SA_CH:], dstb, pa, SB_STEPS)
    new_v = _tc_combine(node_feat, pb)
    return ne_buf, new_v, state_feat


# speedup vs baseline: 1.5556x; 1.3045x over previous
import jax
import jax.numpy as jnp
from jax import lax
from jax.experimental import pallas as pl
from jax.experimental.pallas import tpu as pltpu
from jax.experimental.pallas import tpu_sc as plsc

N = 10000
E = 320000
D = 128

NC = 2
NS = 16
NW = NC * NS

CH = 5
ECH = E // CH

GCHUNK = 80
SCHUNK = 80
GSTEPS = ECH // NW // GCHUNK
SA_CH = 4
SA_STEPS = SA_CH * ECH // NW // SCHUNK
SB_STEPS = ECH // NW // SCHUNK

NP = 10240
ROWS_PER_SUB = NP // NS

BLK = 2000
NBLK = ECH // BLK

_f32 = jnp.float32


def _gather_body(table, sidx3, didx3, vi_hbm, vj_hbm,
                 idx_vs, idx_vd, rows0, rows1, sg0, sg1, sw0, sw1):
    c = lax.axis_index("c")
    s = lax.axis_index("s")
    wid = c * NS + s
    base = wid * GSTEPS * GCHUNK
    pltpu.sync_copy(sidx3.at[wid], idx_vs)
    pltpu.sync_copy(didx3.at[wid], idx_vd)

    def half(idx_v, out_hbm):
        def pair(j, carry):
            k0 = 2 * j
            k1 = k0 + 1
            g0 = pltpu.async_copy(table.at[idx_v.at[k0]], rows0, sg0)
            g1 = pltpu.async_copy(table.at[idx_v.at[k1]], rows1, sg1)
            g0.wait()
            w0 = pltpu.async_copy(
                rows0, out_hbm.at[pl.ds(base + k0 * GCHUNK, GCHUNK)], sw0)
            g1.wait()
            w1 = pltpu.async_copy(
                rows1, out_hbm.at[pl.ds(base + k1 * GCHUNK, GCHUNK)], sw1)
            w0.wait()
            w1.wait()
            return carry

        lax.fori_loop(0, GSTEPS // 2, pair, 0)
        if GSTEPS % 2 == 1:
            kt = GSTEPS - 1
            pltpu.async_copy(table.at[idx_v.at[kt]], rows0, sg0).wait()
            pltpu.sync_copy(
                rows0, out_hbm.at[pl.ds(base + kt * GCHUNK, GCHUNK)])

    half(idx_vs, vi_hbm)
    half(idx_vd, vj_hbm)


def _sc_gather(node_feat, sidx3, didx3):
    return pl.kernel(
        _gather_body,
        out_type=[jax.ShapeDtypeStruct((ECH, D), _f32),
                  jax.ShapeDtypeStruct((ECH, D), _f32)],
        mesh=plsc.VectorSubcoreMesh(core_axis_name="c", subcore_axis_name="s"),
        scratch_types=[
            pltpu.VMEM((GSTEPS, GCHUNK), jnp.int32),
            pltpu.VMEM((GSTEPS, GCHUNK), jnp.int32),
            pltpu.VMEM((GCHUNK, D), _f32),
            pltpu.VMEM((GCHUNK, D), _f32),
            pltpu.SemaphoreType.DMA,
            pltpu.SemaphoreType.DMA,
            pltpu.SemaphoreType.DMA,
            pltpu.SemaphoreType.DMA,
        ],
    )(node_feat, sidx3, didx3)


def _make_scatter_body(steps, nmess, mess_rows_per_worker, zero_init):
    pairs = steps // 2
    has_tail = steps % 2 == 1

    def body(*refs):
        mess_refs = refs[:nmess]
        if zero_init:
            (dst3, out_hbm, idx_v, rows0, rows1, acc,
             sl0, sl1, ss0, ss1, zbuf) = refs[nmess:]
        else:
            (dst3, init, out_hbm, idx_v, rows0, rows1, acc,
             sl0, sl1, ss0, ss1) = refs[nmess:]
        c = lax.axis_index("c")
        s = lax.axis_index("s")

        if zero_init:
            zv = jnp.zeros((16,), _f32)

            def zrow(r, carry):
                for cc in range(D // 16):
                    zbuf[r, pl.ds(cc * 16, 16)] = zv
                return carry

            lax.fori_loop(0, SCHUNK, zrow, 0)

            def zcp(t, carry):
                pltpu.sync_copy(
                    zbuf,
                    acc.at[pl.ds(s * ROWS_PER_SUB + t * SCHUNK, SCHUNK)])
                return carry

            lax.fori_loop(0, ROWS_PER_SUB // SCHUNK, zcp, 0)
        else:
            @pl.when(s == 0)
            def _init():
                pltpu.sync_copy(init.at[c], acc)

        plsc.subcore_barrier()

        wid = c * NS + s
        pltpu.sync_copy(dst3.at[wid], idx_v)

        def accum(mref, lbase):
            def pair(j, carry):
                k0 = 2 * j
                k1 = k0 + 1
                l0 = pltpu.async_copy(
                    mref.at[pl.ds(lbase + k0 * SCHUNK, SCHUNK)], rows0, sl0)
                l1 = pltpu.async_copy(
                    mref.at[pl.ds(lbase + k1 * SCHUNK, SCHUNK)], rows1, sl1)
                l0.wait()
                s0 = pltpu.async_copy(rows0, acc.at[idx_v.at[k0]], ss0,
                                      add=True)
                l1.wait()
                s1 = pltpu.async_copy(rows1, acc.at[idx_v.at[k1]], ss1,
                                      add=True)
                s0.wait()
                s1.wait()
                return carry

            lax.fori_loop(0, pairs, pair, 0)
            if has_tail:
                kt = steps - 1
                pltpu.sync_copy(
                    mref.at[pl.ds(lbase + kt * SCHUNK, SCHUNK)], rows0)
                pltpu.sync_copy(rows0, acc.at[idx_v.at[kt]], add=True)

        if nmess == 1:
            accum(mess_refs[0], wid * mess_rows_per_worker)
        else:
            wpm = NW // nmess
            sub = wid // wpm
            lbase = (wid % wpm) * mess_rows_per_worker
            for mi in range(nmess):
                @pl.when(sub == mi)
                def _go(mi=mi):
                    accum(mess_refs[mi], lbase)

        plsc.subcore_barrier()
        rbase = s * ROWS_PER_SUB
        pltpu.sync_copy(acc.at[pl.ds(rbase, ROWS_PER_SUB)],
                        out_hbm.at[c, pl.ds(rbase, ROWS_PER_SUB)])

    return body


def _sc_scatter(mess_list, dst3, init, steps):
    nmess = len(mess_list)
    rows_per_worker = nmess * ECH // NW
    zero_init = init is None
    scratch = [
        pltpu.VMEM((steps, SCHUNK), jnp.int32),
        pltpu.VMEM((SCHUNK, D), _f32),
        pltpu.VMEM((SCHUNK, D), _f32),
        pltpu.VMEM_SHARED((NP, D), _f32),
        pltpu.SemaphoreType.DMA,
        pltpu.SemaphoreType.DMA,
        pltpu.SemaphoreType.DMA,
        pltpu.SemaphoreType.DMA,
    ]
    args = list(mess_list) + [dst3]
    if zero_init:
        scratch.append(pltpu.VMEM((SCHUNK, D), _f32))
    else:
        args.append(init)
    return pl.kernel(
        _make_scatter_body(steps, nmess, rows_per_worker, zero_init),
        out_type=jax.ShapeDtypeStruct((NC, NP, D), _f32),
        mesh=plsc.VectorSubcoreMesh(core_axis_name="c", subcore_axis_name="s"),
        scratch_types=scratch,
    )(*args)


def _edge_body(vi, vj, ef, rbf, wsrc, wdst, wee, wen, w2, bias, rbfw,
               new_e, mess):
    f32 = jnp.float32
    efb = ef[:]
    pvi = jnp.dot(vi[:], wsrc[:], preferred_element_type=f32)
    pvj = jnp.dot(vj[:], wdst[:], preferred_element_type=f32)
    basep = pvi + pvj
    pe = jnp.dot(efb, wee[:], preferred_element_type=f32)
    r = jnp.dot(rbf[:], rbfw[:], preferred_element_type=f32)

    e_h1 = jax.nn.silu(basep[:, 0:D] + pe[:, 0:D] + bias[0])
    e_g1 = jax.nn.silu(basep[:, D:2 * D] + pe[:, D:2 * D] + bias[2])
    e_h2 = jax.nn.silu(jnp.dot(e_h1, w2[0], preferred_element_type=f32)
                       + bias[1])
    e_g = jax.nn.sigmoid(jnp.dot(e_g1, w2[1], preferred_element_type=f32)
                         + bias[3])
    ne = efb + e_h2 * e_g * r[:, 0:D]
    new_e[:] = ne

    pne = jnp.dot(ne, wen[:], preferred_element_type=f32)
    n_h1 = jax.nn.silu(basep[:, 2 * D:3 * D] + pne[:, 0:D] + bias[4])
    n_g1 = jax.nn.silu(basep[:, 3 * D:4 * D] + pne[:, D:2 * D] + bias[6])
    n_h2 = jax.nn.silu(jnp.dot(n_h1, w2[2], preferred_element_type=f32)
                       + bias[5])
    n_g = jax.nn.sigmoid(jnp.dot(n_g1, w2[3], preferred_element_type=f32)
                         + bias[7])
    mess[:] = n_h2 * n_g * r[:, D:2 * D]


def _edge_body_alias(ne_in, vi, vj, ef, rbf, wsrc, wdst, wee, wen, w2,
                     bias, rbfw, new_e, mess):
    del ne_in
    _edge_body(vi, vj, ef, rbf, wsrc, wdst, wee, wen, w2, bias, rbfw,
               new_e, mess)


def _weight_specs():
    whole2 = lambda i: (0, 0)
    whole3 = lambda i: (0, 0, 0)
    return [
        pl.BlockSpec((D, 4 * D), whole2),
        pl.BlockSpec((D, 4 * D), whole2),
        pl.BlockSpec((D, 2 * D), whole2),
        pl.BlockSpec((D, 2 * D), whole2),
        pl.BlockSpec((4, D, D), whole3),
        pl.BlockSpec((8, D), whole2),
        pl.BlockSpec((9, 2 * D), whole2),
    ]


def _tc_edge_chunk(ne_buf, vi, vj, ef, rbfp, weights, ch):
    row = lambda i: (i, 0)
    rowc = lambda i, ch=ch: (ch * NBLK + i, 0)
    first = ne_buf is None
    body = _edge_body if first else _edge_body_alias
    in_specs = [
        pl.BlockSpec((BLK, D), row),
        pl.BlockSpec((BLK, D), row),
        pl.BlockSpec((BLK, D), rowc),
        pl.BlockSpec((BLK, 9), rowc),
    ] + _weight_specs()
    args = [vi, vj, ef, rbfp] + list(weights)
    aliases = {}
    if not first:
        in_specs = [pl.BlockSpec(memory_space=pltpu.MemorySpace.HBM)] \
            + in_specs
        args = [ne_buf] + args
        aliases = {0: 0}
    return pl.pallas_call(
        body,
        grid=(NBLK,),
        in_specs=in_specs,
        out_specs=[
            pl.BlockSpec((BLK, D), rowc),
            pl.BlockSpec((BLK, D), row),
        ],
        out_shape=[
            jax.ShapeDtypeStruct((E, D), _f32),
            jax.ShapeDtypeStruct((ECH, D), _f32),
        ],
        input_output_aliases=aliases,
        compiler_params=pltpu.CompilerParams(
            dimension_semantics=("arbitrary",)),
    )(*args)


def _combine_body(nf, p, out):
    out[:] = nf[:] + p[0] + p[1]


def _tc_combine(node_feat, partials):
    blk = 1000
    return pl.pallas_call(
        _combine_body,
        grid=(N // blk,),
        in_specs=[
            pl.BlockSpec((blk, D), lambda i: (i, 0)),
            pl.BlockSpec((NC, blk, D), lambda i: (0, i, 0)),
        ],
        out_specs=pl.BlockSpec((blk, D), lambda i: (i, 0)),
        out_shape=jax.ShapeDtypeStruct((N, D), _f32),
    )(node_feat, partials)


def kernel(node_feat, edge_feat, rbf, state_feat, edge_index,
           ew1, eb1, ew2, eb2, egw1, egb1, egw2, egb2, edge_rbf_w,
           nw1, nb1, nw2, nb2, ngw1, ngb1, ngw2, ngb2, node_rbf_w):
    src = edge_index[0].astype(jnp.int32)
    dst = edge_index[1].astype(jnp.int32)
    srcr = src.reshape(CH, NW, GSTEPS, GCHUNK)
    dstr = dst.reshape(CH, NW, GSTEPS, GCHUNK)

    wsrc = jnp.concatenate(
        [ew1[:D], egw1[:D], nw1[:D], ngw1[:D]], axis=1)
    wdst = jnp.concatenate(
        [ew1[D:2 * D], egw1[D:2 * D], nw1[D:2 * D], ngw1[D:2 * D]], axis=1)
    wee = jnp.concatenate([ew1[2 * D:], egw1[2 * D:]], axis=1)
    wen = jnp.concatenate([nw1[2 * D:], ngw1[2 * D:]], axis=1)
    w2 = jnp.stack([ew2, egw2, nw2, ngw2])
    bias = jnp.stack([eb1, eb2, egb1, egb2, nb1, nb2, ngb1, ngb2])
    rbfw = jnp.concatenate([edge_rbf_w, node_rbf_w], axis=1)
    weights = (wsrc, wdst, wee, wen, w2, bias, rbfw)

    ne_buf = None
    mess_chunks = []
    for ch in range(CH):
        vi, vj = _sc_gather(node_feat, srcr[ch], dstr[ch])
        ne_buf, m = _tc_edge_chunk(ne_buf, vi, vj,
                                   edge_feat, rbf, weights, ch)
        mess_chunks.append(m)

    dsta = dst[:SA_CH * ECH].reshape(NW, SA_STEPS, SCHUNK)
    dstb = dst[SA_CH * ECH:].reshape(NW, SB_STEPS, SCHUNK)
    pa = _sc_scatter(mess_chunks[:SA_CH], dsta, None, SA_STEPS)
    pb = _sc_scatter(mess_chunks[SA_CH:], dstb, pa, SB_STEPS)
    new_v = _tc_combine(node_feat, pb)
    return ne_buf, new_v, state_feat


# speedup vs baseline: 1.8405x; 1.1832x over previous
import jax
import jax.numpy as jnp
from jax import lax
from jax.experimental import pallas as pl
from jax.experimental.pallas import tpu as pltpu
from jax.experimental.pallas import tpu_sc as plsc

N = 10000
E = 320000
D = 128

NC = 2
NS = 16
NW = NC * NS

CH = 5
ECH = E // CH

GCHUNK = 80
SCHUNK = 80
GSTEPS = ECH // NW // GCHUNK
SA_CH = 4
SA_STEPS = SA_CH * ECH // NW // SCHUNK
SB_STEPS = ECH // NW // SCHUNK

NP = 10240
ROWS_PER_SUB = NP // NS

BLK = 2560
NBLK = ECH // BLK

_f32 = jnp.float32


def _gather_body(table, sidx3, didx3, vi_hbm, vj_hbm,
                 idx_vs, idx_vd, rows0, rows1, sg0, sg1, sw0, sw1):
    c = lax.axis_index("c")
    s = lax.axis_index("s")
    wid = c * NS + s
    base = wid * GSTEPS * GCHUNK
    pltpu.sync_copy(sidx3.at[wid], idx_vs)
    pltpu.sync_copy(didx3.at[wid], idx_vd)

    def half(idx_v, out_hbm):
        def pair(j, carry):
            k0 = 2 * j
            k1 = k0 + 1
            g0 = pltpu.async_copy(table.at[idx_v.at[k0]], rows0, sg0)
            g1 = pltpu.async_copy(table.at[idx_v.at[k1]], rows1, sg1)
            g0.wait()
            w0 = pltpu.async_copy(
                rows0, out_hbm.at[pl.ds(base + k0 * GCHUNK, GCHUNK)], sw0)
            g1.wait()
            w1 = pltpu.async_copy(
                rows1, out_hbm.at[pl.ds(base + k1 * GCHUNK, GCHUNK)], sw1)
            w0.wait()
            w1.wait()
            return carry

        lax.fori_loop(0, GSTEPS // 2, pair, 0)
        if GSTEPS % 2 == 1:
            kt = GSTEPS - 1
            pltpu.async_copy(table.at[idx_v.at[kt]], rows0, sg0).wait()
            pltpu.sync_copy(
                rows0, out_hbm.at[pl.ds(base + kt * GCHUNK, GCHUNK)])

    half(idx_vs, vi_hbm)
    half(idx_vd, vj_hbm)


def _sc_gather(node_feat, sidx3, didx3):
    return pl.kernel(
        _gather_body,
        out_type=[jax.ShapeDtypeStruct((ECH, D), _f32),
                  jax.ShapeDtypeStruct((ECH, D), _f32)],
        mesh=plsc.VectorSubcoreMesh(core_axis_name="c", subcore_axis_name="s"),
        scratch_types=[
            pltpu.VMEM((GSTEPS, GCHUNK), jnp.int32),
            pltpu.VMEM((GSTEPS, GCHUNK), jnp.int32),
            pltpu.VMEM((GCHUNK, D), _f32),
            pltpu.VMEM((GCHUNK, D), _f32),
            pltpu.SemaphoreType.DMA,
            pltpu.SemaphoreType.DMA,
            pltpu.SemaphoreType.DMA,
            pltpu.SemaphoreType.DMA,
        ],
    )(node_feat, sidx3, didx3)


def _make_scatter_body(steps, nmess, mess_rows_per_worker, zero_init):
    pairs = steps // 2
    has_tail = steps % 2 == 1

    def body(*refs):
        mess_refs = refs[:nmess]
        if zero_init:
            (dst3, out_hbm, idx_v, rows0, rows1, acc,
             sl0, sl1, ss0, ss1, zbuf) = refs[nmess:]
        else:
            (dst3, init, out_hbm, idx_v, rows0, rows1, acc,
             sl0, sl1, ss0, ss1) = refs[nmess:]
        c = lax.axis_index("c")
        s = lax.axis_index("s")

        if zero_init:
            zv = jnp.zeros((16,), _f32)

            def zrow(r, carry):
                for cc in range(D // 16):
                    zbuf[r, pl.ds(cc * 16, 16)] = zv
                return carry

            lax.fori_loop(0, SCHUNK, zrow, 0)

            def zcp(t, carry):
                pltpu.sync_copy(
                    zbuf,
                    acc.at[pl.ds(s * ROWS_PER_SUB + t * SCHUNK, SCHUNK)])
                return carry

            lax.fori_loop(0, ROWS_PER_SUB // SCHUNK, zcp, 0)
        else:
            @pl.when(s == 0)
            def _init():
                pltpu.sync_copy(init.at[c], acc)

        plsc.subcore_barrier()

        wid = c * NS + s
        pltpu.sync_copy(dst3.at[wid], idx_v)

        def accum(mref, lbase):
            def pair(j, carry):
                k0 = 2 * j
                k1 = k0 + 1
                l0 = pltpu.async_copy(
                    mref.at[pl.ds(lbase + k0 * SCHUNK, SCHUNK)], rows0, sl0)
                l1 = pltpu.async_copy(
                    mref.at[pl.ds(lbase + k1 * SCHUNK, SCHUNK)], rows1, sl1)
                l0.wait()
                s0 = pltpu.async_copy(rows0, acc.at[idx_v.at[k0]], ss0,
                                      add=True)
                l1.wait()
                s1 = pltpu.async_copy(rows1, acc.at[idx_v.at[k1]], ss1,
                                      add=True)
                s0.wait()
                s1.wait()
                return carry

            lax.fori_loop(0, pairs, pair, 0)
            if has_tail:
                kt = steps - 1
                pltpu.sync_copy(
                    mref.at[pl.ds(lbase + kt * SCHUNK, SCHUNK)], rows0)
                pltpu.sync_copy(rows0, acc.at[idx_v.at[kt]], add=True)

        if nmess == 1:
            accum(mess_refs[0], wid * mess_rows_per_worker)
        else:
            wpm = NW // nmess
            sub = wid // wpm
            lbase = (wid % wpm) * mess_rows_per_worker
            for mi in range(nmess):
                @pl.when(sub == mi)
                def _go(mi=mi):
                    accum(mess_refs[mi], lbase)

        plsc.subcore_barrier()
        rbase = s * ROWS_PER_SUB
        pltpu.sync_copy(acc.at[pl.ds(rbase, ROWS_PER_SUB)],
                        out_hbm.at[c, pl.ds(rbase, ROWS_PER_SUB)])

    return body


def _sc_scatter(mess_list, dst3, init, steps):
    nmess = len(mess_list)
    rows_per_worker = nmess * ECH // NW
    zero_init = init is None
    scratch = [
        pltpu.VMEM((steps, SCHUNK), jnp.int32),
        pltpu.VMEM((SCHUNK, D), _f32),
        pltpu.VMEM((SCHUNK, D), _f32),
        pltpu.VMEM_SHARED((NP, D), _f32),
        pltpu.SemaphoreType.DMA,
        pltpu.SemaphoreType.DMA,
        pltpu.SemaphoreType.DMA,
        pltpu.SemaphoreType.DMA,
    ]
    args = list(mess_list) + [dst3]
    if zero_init:
        scratch.append(pltpu.VMEM((SCHUNK, D), _f32))
    else:
        args.append(init)
    return pl.kernel(
        _make_scatter_body(steps, nmess, rows_per_worker, zero_init),
        out_type=jax.ShapeDtypeStruct((NC, NP, D), _f32),
        mesh=plsc.VectorSubcoreMesh(core_axis_name="c", subcore_axis_name="s"),
        scratch_types=scratch,
    )(*args)


def _edge_body(vi, vj, ef, rbf, wsrc, wdst, wee, wen, w2, bias, rbfw,
               new_e, mess):
    f32 = jnp.float32
    efb = ef[:]
    pvi = jnp.dot(vi[:], wsrc[:], preferred_element_type=f32)
    pvj = jnp.dot(vj[:], wdst[:], preferred_element_type=f32)
    basep = pvi + pvj
    pe = jnp.dot(efb, wee[:], preferred_element_type=f32)
    r = lax.dot_general(rbf[:], rbfw[:], (((0,), (0,)), ((), ())),
                        preferred_element_type=f32)

    e_h1 = jax.nn.silu(basep[:, 0:D] + pe[:, 0:D] + bias[0])
    e_g1 = jax.nn.silu(basep[:, D:2 * D] + pe[:, D:2 * D] + bias[2])
    e_h2 = jax.nn.silu(jnp.dot(e_h1, w2[0], preferred_element_type=f32)
                       + bias[1])
    e_g = jax.nn.sigmoid(jnp.dot(e_g1, w2[1], preferred_element_type=f32)
                         + bias[3])
    ne = efb + e_h2 * e_g * r[:, 0:D]
    new_e[:] = ne

    pne = jnp.dot(ne, wen[:], preferred_element_type=f32)
    n_h1 = jax.nn.silu(basep[:, 2 * D:3 * D] + pne[:, 0:D] + bias[4])
    n_g1 = jax.nn.silu(basep[:, 3 * D:4 * D] + pne[:, D:2 * D] + bias[6])
    n_h2 = jax.nn.silu(jnp.dot(n_h1, w2[2], preferred_element_type=f32)
                       + bias[5])
    n_g = jax.nn.sigmoid(jnp.dot(n_g1, w2[3], preferred_element_type=f32)
                         + bias[7])
    mess[:] = n_h2 * n_g * r[:, D:2 * D]


def _edge_body_alias(ne_in, vi, vj, ef, rbf, wsrc, wdst, wee, wen, w2,
                     bias, rbfw, new_e, mess):
    del ne_in
    _edge_body(vi, vj, ef, rbf, wsrc, wdst, wee, wen, w2, bias, rbfw,
               new_e, mess)


def _weight_specs():
    whole2 = lambda i: (0, 0)
    whole3 = lambda i: (0, 0, 0)
    return [
        pl.BlockSpec((D, 4 * D), whole2),
        pl.BlockSpec((D, 4 * D), whole2),
        pl.BlockSpec((D, 2 * D), whole2),
        pl.BlockSpec((D, 2 * D), whole2),
        pl.BlockSpec((4, D, D), whole3),
        pl.BlockSpec((8, D), whole2),
        pl.BlockSpec((9, 2 * D), whole2),
    ]


def _tc_edge_chunk(ne_buf, vi, vj, ef, rbfp, weights, ch):
    row = lambda i: (i, 0)
    rowc = lambda i, ch=ch: (ch * NBLK + i, 0)
    colc = lambda i, ch=ch: (0, ch * NBLK + i)
    first = ne_buf is None
    body = _edge_body if first else _edge_body_alias
    in_specs = [
        pl.BlockSpec((BLK, D), row),
        pl.BlockSpec((BLK, D), row),
        pl.BlockSpec((BLK, D), rowc),
        pl.BlockSpec((9, BLK), colc),
    ] + _weight_specs()
    args = [vi, vj, ef, rbfp] + list(weights)
    aliases = {}
    if not first:
        in_specs = [pl.BlockSpec(memory_space=pltpu.MemorySpace.HBM)] \
            + in_specs
        args = [ne_buf] + args
        aliases = {0: 0}
    return pl.pallas_call(
        body,
        grid=(NBLK,),
        in_specs=in_specs,
        out_specs=[
            pl.BlockSpec((BLK, D), rowc),
            pl.BlockSpec((BLK, D), row),
        ],
        out_shape=[
            jax.ShapeDtypeStruct((E, D), _f32),
            jax.ShapeDtypeStruct((ECH, D), _f32),
        ],
        input_output_aliases=aliases,
        compiler_params=pltpu.CompilerParams(
            dimension_semantics=("arbitrary",)),
    )(*args)


def _combine_body(nf, p, out):
    out[:] = nf[:] + p[0] + p[1]


def _tc_combine(node_feat, partials):
    blk = 1000
    return pl.pallas_call(
        _combine_body,
        grid=(N // blk,),
        in_specs=[
            pl.BlockSpec((blk, D), lambda i: (i, 0)),
            pl.BlockSpec((NC, blk, D), lambda i: (0, i, 0)),
        ],
        out_specs=pl.BlockSpec((blk, D), lambda i: (i, 0)),
        out_shape=jax.ShapeDtypeStruct((N, D), _f32),
    )(node_feat, partials)


def kernel(node_feat, edge_feat, rbf, state_feat, edge_index,
           ew1, eb1, ew2, eb2, egw1, egb1, egw2, egb2, edge_rbf_w,
           nw1, nb1, nw2, nb2, ngw1, ngb1, ngw2, ngb2, node_rbf_w):
    src = edge_index[0].astype(jnp.int32)
    dst = edge_index[1].astype(jnp.int32)
    srcr = src.reshape(CH, NW, GSTEPS, GCHUNK)
    dstr = dst.reshape(CH, NW, GSTEPS, GCHUNK)

    wsrc = jnp.concatenate(
        [ew1[:D], egw1[:D], nw1[:D], ngw1[:D]], axis=1)
    wdst = jnp.concatenate(
        [ew1[D:2 * D], egw1[D:2 * D], nw1[D:2 * D], ngw1[D:2 * D]], axis=1)
    wee = jnp.concatenate([ew1[2 * D:], egw1[2 * D:]], axis=1)
    wen = jnp.concatenate([nw1[2 * D:], ngw1[2 * D:]], axis=1)
    w2 = jnp.stack([ew2, egw2, nw2, ngw2])
    bias = jnp.stack([eb1, eb2, egb1, egb2, nb1, nb2, ngb1, ngb2])
    rbfw = jnp.concatenate([edge_rbf_w, node_rbf_w], axis=1)
    rbft = rbf.T
    weights = (wsrc, wdst, wee, wen, w2, bias, rbfw)

    ne_buf = None
    mess_chunks = []
    for ch in range(CH):
        vi, vj = _sc_gather(node_feat, srcr[ch], dstr[ch])
        ne_buf, m = _tc_edge_chunk(ne_buf, vi, vj,
                                   edge_feat, rbft, weights, ch)
        mess_chunks.append(m)

    dsta = dst[:SA_CH * ECH].reshape(NW, SA_STEPS, SCHUNK)
    dstb = dst[SA_CH * ECH:].reshape(NW, SB_STEPS, SCHUNK)
    pa = _sc_scatter(mess_chunks[:SA_CH], dsta, None, SA_STEPS)
    pb = _sc_scatter(mess_chunks[SA_CH:], dstb, pa, SB_STEPS)
    new_v = _tc_combine(node_feat, pb)
    return ne_buf, new_v, state_feat
